# Initial kernel scaffold; baseline (speedup 1.0000x reference)
import functools

import jax
import jax.numpy as jnp
from jax import lax
from jax.experimental import pallas as pl
from jax.experimental.pallas import tpu as pltpu
from jax.experimental.pallas import tpu_sc as plsc

N_FULL = 10000
N_TRUNC = 2500
E = 160000
D_FEAT = 128
BE = 2
F = BE * D_FEAT
NW = 32
CHUNK = 64
NB = 40


def _make_spmm(rows_per_tile):
    mesh = plsc.VectorSubcoreMesh(core_axis_name="c", subcore_axis_name="s")

    @functools.partial(
        pl.kernel,
        out_type=jax.ShapeDtypeStruct((NW * rows_per_tile, F), jnp.float32),
        mesh=mesh,
        scratch_types=[
            pltpu.VMEM((rows_per_tile, F), jnp.float32),
            pltpu.VMEM((CHUNK, F), jnp.float32),
            pltpu.VMEM((CHUNK,), jnp.int32),
            pltpu.VMEM((CHUNK,), jnp.int32),
            pltpu.VMEM((CHUNK,), jnp.float32),
            pltpu.VMEM((NB,), jnp.int32),
            pltpu.SemaphoreType.DMA,
        ],
    )
    def spmm(table, src, dst, w, bounds, out, acc_v, rows_v, src_v, dst_v,
             w_v, b_v, sem):
        wid = lax.axis_index("c") * 16 + lax.axis_index("s")
        row0 = wid * rows_per_tile

        pltpu.sync_copy(bounds, b_v)
        e0 = b_v[wid]
        e1 = b_v[wid + 1]

        zero = jnp.zeros((16,), jnp.float32)

        def zrow(r, carry):
            for k in range(F // 16):
                acc_v[r, pl.ds(k * 16, 16)] = zero
            return carry

        lax.fori_loop(0, rows_per_tile, zrow, 0)

        a0 = (e0 // CHUNK) * CHUNK
        nch = (e1 - a0 + CHUNK - 1) // CHUNK

        def chunk_body(c, carry):
            base = a0 + c * CHUNK
            pltpu.sync_copy(src.at[pl.ds(base, CHUNK)], src_v)
            pltpu.sync_copy(dst.at[pl.ds(base, CHUNK)], dst_v)
            pltpu.sync_copy(w.at[pl.ds(base, CHUNK)], w_v)
            pltpu.async_copy(table.at[src_v], rows_v, sem).wait()
            jlo = jnp.maximum(e0 - base, 0)
            jhi = jnp.minimum(e1 - base, CHUNK)

            def edge_body(j, ecarry):
                dl = dst_v[j] - row0
                wj = w_v[j]
                for k in range(F // 16):
                    plsc.addupdate(
                        acc_v.at[dl, pl.ds(k * 16, 16)],
                        wj * rows_v[j, pl.ds(k * 16, 16)],
                    )
                return ecarry

            lax.fori_loop(jlo, jhi, edge_body, 0)
            return carry

        lax.fori_loop(0, nch, chunk_body, 0)

        pltpu.sync_copy(acc_v, out.at[pl.ds(row0, rows_per_tile)])

    return spmm


ROWS_D = (N_TRUNC + NW - 1) // NW
ROWS_U = (N_FULL + NW - 1) // NW

_spmm_down = _make_spmm(ROWS_D)
_spmm_up = _make_spmm(ROWS_U)


def _bounds(dst, rows_per_tile):
    edges = jnp.arange(NW + 1, dtype=jnp.int32) * rows_per_tile
    b = jnp.searchsorted(dst, edges, side="left").astype(jnp.int32)
    return jnp.concatenate([b, jnp.full((NB - NW - 1,), E, jnp.int32)])


def kernel(x, down_src, down_dst, down_w, up_src, up_dst, up_w):
    batch, _, ens, _, _ = x.shape
    xb = x[:, -1].reshape(batch * ens, N_FULL, D_FEAT)
    xt = jnp.transpose(xb, (1, 0, 2)).reshape(N_FULL, F)

    bd = _bounds(down_dst, ROWS_D)
    bu = _bounds(up_dst, ROWS_U)

    h = _spmm_down(xt, down_src, down_dst, down_w, bd)
    y = _spmm_up(h, up_src, up_dst, up_w, bu)

    y = y[:N_FULL].reshape(N_FULL, BE, D_FEAT)
    return jnp.transpose(y, (1, 0, 2)).reshape(batch, ens, N_FULL, D_FEAT)


# speedup vs baseline: 16.9689x; 16.9689x over previous
import functools

import jax
import jax.numpy as jnp
from jax import lax
from jax.experimental import pallas as pl
from jax.experimental.pallas import tpu as pltpu
from jax.experimental.pallas import tpu_sc as plsc

N_FULL = 10000
N_TRUNC = 2500
E = 160000
D_FEAT = 128
BE = 2
F = BE * D_FEAT
NW = 32
CHUNK = 64
NB = 48
CPAD = CHUNK + 16


def _make_spmm(rows_per_tile):
    mesh = plsc.VectorSubcoreMesh(core_axis_name="c", subcore_axis_name="s")

    @functools.partial(
        pl.kernel,
        out_type=jax.ShapeDtypeStruct((NW * rows_per_tile, F), jnp.float32),
        mesh=mesh,
        scratch_types=[
            pltpu.VMEM((rows_per_tile, F), jnp.float32),
            pltpu.VMEM((CHUNK, F), jnp.float32),
            pltpu.VMEM((CHUNK,), jnp.int32),
            pltpu.VMEM((CPAD,), jnp.int32),
            pltpu.VMEM((CPAD,), jnp.float32),
            pltpu.VMEM((NB,), jnp.int32),
            pltpu.SemaphoreType.DMA,
        ],
        compiler_params=pltpu.CompilerParams(use_tc_tiling_on_sc=False),
    )
    def spmm(table, src, dst, w, bounds, out, acc_v, rows_v, src_v, dst_v,
             w_v, b_v, sem):
        wid = lax.axis_index("c") * 16 + lax.axis_index("s")
        row0 = wid * rows_per_tile

        pltpu.sync_copy(bounds, b_v)
        bb = b_v[pl.ds(wid, 16)]
        e0 = bb[0]
        e1 = bb[1]

        zero = jnp.zeros((16,), jnp.float32)

        def zrow(r, carry):
            for k in range(F // 16):
                acc_v[r, pl.ds(k * 16, 16)] = zero
            return carry

        lax.fori_loop(0, rows_per_tile, zrow, 0)

        a0 = (e0 // CHUNK) * CHUNK
        nch = (e1 - a0 + CHUNK - 1) // CHUNK

        def chunk_body(c, carry):
            base = a0 + c * CHUNK
            pltpu.sync_copy(src.at[pl.ds(base, CHUNK)], src_v)
            pltpu.sync_copy(dst.at[pl.ds(base, CHUNK)], dst_v.at[pl.ds(0, CHUNK)])
            pltpu.sync_copy(w.at[pl.ds(base, CHUNK)], w_v.at[pl.ds(0, CHUNK)])
            pltpu.async_copy(table.at[src_v], rows_v, sem).wait()
            jlo = jnp.maximum(e0 - base, 0)
            jhi = jnp.minimum(e1 - base, CHUNK)

            def edge_body(j, ecarry):
                dl = dst_v[pl.ds(j, 16)][0] - row0
                wj = w_v[pl.ds(j, 16)][0]
                for k in range(F // 16):
                    plsc.addupdate(
                        acc_v.at[dl, pl.ds(k * 16, 16)],
                        wj * rows_v[j, pl.ds(k * 16, 16)],
                    )
                return ecarry

            lax.fori_loop(jlo, jhi, edge_body, 0)
            return carry

        lax.fori_loop(0, nch, chunk_body, 0)

        pltpu.sync_copy(acc_v, out.at[pl.ds(row0, rows_per_tile)])

    return spmm


def _round8(v):
    return (v + 7) // 8 * 8


ROWS_D = _round8((N_TRUNC + NW - 1) // NW)
ROWS_U = _round8((N_FULL + NW - 1) // NW)

_spmm_down = _make_spmm(ROWS_D)
_spmm_up = _make_spmm(ROWS_U)


def _bounds(dst, rows_per_tile):
    edges = jnp.arange(NW + 1, dtype=jnp.int32) * rows_per_tile
    b = jnp.searchsorted(dst, edges, side="left").astype(jnp.int32)
    return jnp.concatenate([b, jnp.full((NB - NW - 1,), E, jnp.int32)])


def kernel(x, down_src, down_dst, down_w, up_src, up_dst, up_w):
    batch, _, ens, _, _ = x.shape
    xb = x[:, -1].reshape(batch * ens, N_FULL, D_FEAT)
    xt = jnp.transpose(xb, (1, 0, 2)).reshape(N_FULL, F)

    bd = _bounds(down_dst, ROWS_D)
    bu = _bounds(up_dst, ROWS_U)

    h = _spmm_down(xt, down_src, down_dst, down_w, bd)
    y = _spmm_up(h, up_src, up_dst, up_w, bu)

    y = y[:N_FULL].reshape(N_FULL, BE, D_FEAT)
    return jnp.transpose(y, (1, 0, 2)).reshape(batch, ens, N_FULL, D_FEAT)


# speedup vs baseline: 23.7538x; 1.3998x over previous
import functools

import jax
import jax.numpy as jnp
from jax import lax
from jax.experimental import pallas as pl
from jax.experimental.pallas import tpu as pltpu
from jax.experimental.pallas import tpu_sc as plsc

N_FULL = 10000
N_TRUNC = 2500
E = 160000
D_FEAT = 128
BE = 2
F = BE * D_FEAT
NW = 32
CHUNK = 64
NB = 48


def _make_spmm(rows_per_tile):
    mesh = plsc.VectorSubcoreMesh(core_axis_name="c", subcore_axis_name="s")

    @functools.partial(
        pl.kernel,
        out_type=jax.ShapeDtypeStruct((NW * rows_per_tile, F), jnp.float32),
        mesh=mesh,
        scratch_types=[
            pltpu.VMEM((rows_per_tile, F), jnp.float32),
            [pltpu.VMEM((CHUNK, F), jnp.float32)] * 2,
            [pltpu.VMEM((CHUNK,), jnp.int32)] * 2,
            [pltpu.VMEM((CHUNK,), jnp.int32)] * 2,
            [pltpu.VMEM((CHUNK,), jnp.float32)] * 2,
            pltpu.VMEM((NB,), jnp.int32),
            [pltpu.SemaphoreType.DMA] * 2,
            [pltpu.SemaphoreType.DMA] * 2,
        ],
    )
    def spmm(table, src, dst, w, bounds, out, acc_v, rows_v, src_v, dst_v,
             w_v, b_v, semS, semG):
        wid = lax.axis_index("c") * 16 + lax.axis_index("s")
        row0 = wid * rows_per_tile

        pltpu.sync_copy(bounds, b_v)
        bb = b_v[pl.ds(wid, 16)]
        e0 = bb[0]
        e1 = bb[1]

        a0 = (e0 // CHUNK) * CHUNK
        nch = (e1 - a0 + CHUNK - 1) // CHUNK

        def stage_start(c, s):
            base = a0 + c * CHUNK
            pltpu.async_copy(src.at[pl.ds(base, CHUNK)], src_v[s], semS[s])
            pltpu.async_copy(dst.at[pl.ds(base, CHUNK)], dst_v[s], semS[s])
            pltpu.async_copy(w.at[pl.ds(base, CHUNK)], w_v[s], semS[s])

        def stage_wait(s):
            pltpu.make_async_copy(src.at[pl.ds(0, CHUNK)], src_v[s], semS[s]).wait()
            pltpu.make_async_copy(dst.at[pl.ds(0, CHUNK)], dst_v[s], semS[s]).wait()
            pltpu.make_async_copy(w.at[pl.ds(0, CHUNK)], w_v[s], semS[s]).wait()

        def gather_start(s):
            pltpu.async_copy(table.at[src_v[s]], rows_v[s], semG[s])

        def gather_wait(s):
            pltpu.make_async_copy(table.at[src_v[s]], rows_v[s], semG[s]).wait()

        def compute(c, s):
            base = a0 + c * CHUNK
            jlo = jnp.maximum(e0 - base, 0)
            jhi = jnp.minimum(e1 - base, CHUNK)
            iota = lax.iota(jnp.int32, 16)

            def group(g, carry):
                lane = iota + g * 16
                m = (lane >= jlo) & (lane < jhi)
                dvec = dst_v[s][pl.ds(g * 16, 16)]
                wvec = w_v[s][pl.ds(g * 16, 16)]
                wm = jnp.where(m, wvec, 0.0)
                dl = jnp.clip(dvec - row0, 0, rows_per_tile - 1)
                for l in range(16):
                    wj = wm[l]
                    dlj = dl[l]
                    r = g * 16 + l
                    for k in range(F // 16):
                        plsc.addupdate(
                            acc_v.at[dlj, pl.ds(k * 16, 16)],
                            wj * rows_v[s][r, pl.ds(k * 16, 16)],
                        )
                return carry

            lax.fori_loop(0, CHUNK // 16, group, 0)

        @pl.when(nch > 0)
        def _():
            stage_start(0, 0)

        zero = jnp.zeros((16,), jnp.float32)

        def zrow(r, carry):
            for k in range(F // 16):
                acc_v[r, pl.ds(k * 16, 16)] = zero
            return carry

        lax.fori_loop(0, rows_per_tile, zrow, 0)

        @pl.when(nch > 0)
        def _():
            stage_wait(0)
            gather_start(0)

        @pl.when(nch > 1)
        def _():
            stage_start(1, 1)

        def pair_body(i, carry):
            c0 = 2 * i
            c1 = c0 + 1
            c2 = c0 + 2
            c3 = c0 + 3

            @pl.when(c1 < nch)
            def _():
                stage_wait(1)
                gather_start(1)

            gather_wait(0)
            compute(c0, 0)

            @pl.when(c2 < nch)
            def _():
                stage_start(c2, 0)
                stage_wait(0)
                gather_start(0)

            @pl.when(c1 < nch)
            def _():
                gather_wait(1)
                compute(c1, 1)

            @pl.when(c3 < nch)
            def _():
                stage_start(c3, 1)

            return carry

        lax.fori_loop(0, (nch + 1) // 2, pair_body, 0)

        pltpu.sync_copy(acc_v, out.at[pl.ds(row0, rows_per_tile)])

    return spmm


def _round8(v):
    return (v + 7) // 8 * 8


ROWS_D = _round8((N_TRUNC + NW - 1) // NW)
ROWS_U = _round8((N_FULL + NW - 1) // NW)

_spmm_down = _make_spmm(ROWS_D)
_spmm_up = _make_spmm(ROWS_U)


def _bounds(dst, rows_per_tile):
    edges = jnp.arange(NW + 1, dtype=jnp.int32) * rows_per_tile
    b = jnp.searchsorted(dst, edges, side="left").astype(jnp.int32)
    return jnp.concatenate([b, jnp.full((NB - NW - 1,), E, jnp.int32)])


def kernel(x, down_src, down_dst, down_w, up_src, up_dst, up_w):
    batch, _, ens, _, _ = x.shape
    xb = x[:, -1].reshape(batch * ens, N_FULL, D_FEAT)
    xt = jnp.transpose(xb, (1, 0, 2)).reshape(N_FULL, F)

    bd = _bounds(down_dst, ROWS_D)
    bu = _bounds(up_dst, ROWS_U)

    h = _spmm_down(xt, down_src, down_dst, down_w, bd)
    y = _spmm_up(h, up_src, up_dst, up_w, bu)

    y = y[:N_FULL].reshape(N_FULL, BE, D_FEAT)
    return jnp.transpose(y, (1, 0, 2)).reshape(batch, ens, N_FULL, D_FEAT)


# speedup vs baseline: 47.6095x; 2.0043x over previous
import functools

import jax
import jax.numpy as jnp
from jax import lax
from jax.experimental import pallas as pl
from jax.experimental.pallas import tpu as pltpu
from jax.experimental.pallas import tpu_sc as plsc

N_FULL = 10000
N_TRUNC = 2500
E = 160000
D_FEAT = 128
BE = 2
F = BE * D_FEAT
NW = 32
CHUNK = 64
NB = 48


def _make_spmm(rows_per_tile):
    mesh = plsc.VectorSubcoreMesh(core_axis_name="c", subcore_axis_name="s")

    @functools.partial(
        pl.kernel,
        out_type=jax.ShapeDtypeStruct((NW * rows_per_tile, F), jnp.float32),
        mesh=mesh,
        scratch_types=[
            pltpu.VMEM((rows_per_tile, F), jnp.float32),
            [pltpu.VMEM((CHUNK, F), jnp.float32)] * 2,
            [pltpu.VMEM((CHUNK,), jnp.int32)] * 2,
            [pltpu.VMEM((CHUNK,), jnp.int32)] * 2,
            [pltpu.VMEM((CHUNK,), jnp.float32)] * 2,
            pltpu.VMEM((NB,), jnp.int32),
            [pltpu.SemaphoreType.DMA] * 2,
            [pltpu.SemaphoreType.DMA] * 2,
        ],
    )
    def spmm(table, src, dst, w, bounds, out, acc_v, rows_v, src_v, dst_v,
             w_v, b_v, semS, semG):
        wid = lax.axis_index("c") * 16 + lax.axis_index("s")
        row0 = wid * rows_per_tile

        pltpu.sync_copy(bounds, b_v)
        bb = b_v[pl.ds(wid, 16)]
        e0 = bb[0]
        e1 = bb[1]

        a0 = (e0 // CHUNK) * CHUNK
        nch = (e1 - a0 + CHUNK - 1) // CHUNK

        def stage_start(c, s):
            base = a0 + c * CHUNK
            pltpu.async_copy(src.at[pl.ds(base, CHUNK)], src_v[s], semS[s])
            pltpu.async_copy(dst.at[pl.ds(base, CHUNK)], dst_v[s], semS[s])
            pltpu.async_copy(w.at[pl.ds(base, CHUNK)], w_v[s], semS[s])

        def stage_wait(s):
            pltpu.make_async_copy(src.at[pl.ds(0, CHUNK)], src_v[s], semS[s]).wait()
            pltpu.make_async_copy(dst.at[pl.ds(0, CHUNK)], dst_v[s], semS[s]).wait()
            pltpu.make_async_copy(w.at[pl.ds(0, CHUNK)], w_v[s], semS[s]).wait()

        def gather_start(s):
            pltpu.async_copy(table.at[src_v[s]], rows_v[s], semG[s])

        def gather_wait(s):
            pltpu.make_async_copy(table.at[src_v[s]], rows_v[s], semG[s]).wait()

        def compute(c, s):
            base = a0 + c * CHUNK
            jlo = jnp.maximum(e0 - base, 0)
            jhi = jnp.minimum(e1 - base, CHUNK)
            iota = lax.iota(jnp.int32, 16)

            def group(g, carry):
                lane = iota + g * 16
                m = (lane >= jlo) & (lane < jhi)
                dvec = dst_v[s][pl.ds(g * 16, 16)]
                wvec = w_v[s][pl.ds(g * 16, 16)]
                wm = jnp.where(m, wvec, 0.0)
                dl = jnp.clip(dvec - row0, 0, rows_per_tile - 1)
                for l in range(16):
                    wj = wm[l]
                    dlj = dl[l]
                    r = g * 16 + l

                    @plsc.parallel_loop(0, F // 16, 1, unroll=8)
                    def _(k, _wj=wj, _dlj=dlj, _r=r):
                        plsc.addupdate(
                            acc_v.at[_dlj, pl.ds(k * 16, 16)],
                            _wj * rows_v[s][_r, pl.ds(k * 16, 16)],
                        )
                return carry

            lax.fori_loop(0, CHUNK // 16, group, 0)

        @pl.when(nch > 0)
        def _():
            stage_start(0, 0)

        zero = jnp.zeros((16,), jnp.float32)

        def zrow(r, carry):
            for k in range(F // 16):
                acc_v[r, pl.ds(k * 16, 16)] = zero
            return carry

        lax.fori_loop(0, rows_per_tile, zrow, 0)

        @pl.when(nch > 0)
        def _():
            stage_wait(0)
            gather_start(0)

        @pl.when(nch > 1)
        def _():
            stage_start(1, 1)

        def pair_body(i, carry):
            c0 = 2 * i
            c1 = c0 + 1
            c2 = c0 + 2
            c3 = c0 + 3

            @pl.when(c1 < nch)
            def _():
                stage_wait(1)
                gather_start(1)

            gather_wait(0)
            compute(c0, 0)

            @pl.when(c2 < nch)
            def _():
                stage_start(c2, 0)
                stage_wait(0)
                gather_start(0)

            @pl.when(c1 < nch)
            def _():
                gather_wait(1)
                compute(c1, 1)

            @pl.when(c3 < nch)
            def _():
                stage_start(c3, 1)

            return carry

        lax.fori_loop(0, (nch + 1) // 2, pair_body, 0)

        pltpu.sync_copy(acc_v, out.at[pl.ds(row0, rows_per_tile)])

    return spmm


def _round8(v):
    return (v + 7) // 8 * 8


ROWS_D = _round8((N_TRUNC + NW - 1) // NW)
ROWS_U = _round8((N_FULL + NW - 1) // NW)

_spmm_down = _make_spmm(ROWS_D)
_spmm_up = _make_spmm(ROWS_U)


def _bounds(dst, rows_per_tile):
    edges = jnp.arange(NW + 1, dtype=jnp.int32) * rows_per_tile
    b = jnp.searchsorted(dst, edges, side="left").astype(jnp.int32)
    return jnp.concatenate([b, jnp.full((NB - NW - 1,), E, jnp.int32)])


def kernel(x, down_src, down_dst, down_w, up_src, up_dst, up_w):
    batch, _, ens, _, _ = x.shape
    xb = x[:, -1].reshape(batch * ens, N_FULL, D_FEAT)
    xt = jnp.transpose(xb, (1, 0, 2)).reshape(N_FULL, F)

    bd = _bounds(down_dst, ROWS_D)
    bu = _bounds(up_dst, ROWS_U)

    h = _spmm_down(xt, down_src, down_dst, down_w, bd)
    y = _spmm_up(h, up_src, up_dst, up_w, bu)

    y = y[:N_FULL].reshape(N_FULL, BE, D_FEAT)
    return jnp.transpose(y, (1, 0, 2)).reshape(batch, ens, N_FULL, D_FEAT)


# speedup vs baseline: 48.9935x; 1.0291x over previous
import functools

import jax
import jax.numpy as jnp
from jax import lax
from jax.experimental import pallas as pl
from jax.experimental.pallas import tpu as pltpu
from jax.experimental.pallas import tpu_sc as plsc

N_FULL = 10000
N_TRUNC = 2500
E = 160000
D_FEAT = 128
BE = 2
F = BE * D_FEAT
NW = 32
NB = 48


def _make_spmm(rows_per_tile, CHUNK):
    mesh = plsc.VectorSubcoreMesh(core_axis_name="c", subcore_axis_name="s")

    @functools.partial(
        pl.kernel,
        out_type=jax.ShapeDtypeStruct((NW * rows_per_tile, F), jnp.float32),
        mesh=mesh,
        scratch_types=[
            pltpu.VMEM((rows_per_tile, F), jnp.float32),
            [pltpu.VMEM((CHUNK, F), jnp.float32)] * 2,
            [pltpu.VMEM((CHUNK,), jnp.int32)] * 2,
            [pltpu.VMEM((CHUNK,), jnp.int32)] * 2,
            [pltpu.VMEM((CHUNK,), jnp.float32)] * 2,
            pltpu.VMEM((NB,), jnp.int32),
            [pltpu.SemaphoreType.DMA] * 2,
            [pltpu.SemaphoreType.DMA] * 2,
        ],
    )
    def spmm(table, src, dst, w, bounds, out, acc_v, rows_v, src_v, dst_v,
             w_v, b_v, semS, semG):
        wid = lax.axis_index("c") * 16 + lax.axis_index("s")
        row0 = wid * rows_per_tile

        pltpu.sync_copy(bounds, b_v)
        bb = b_v[pl.ds(wid, 16)]
        e0 = bb[0]
        e1 = bb[1]

        a0 = (e0 // CHUNK) * CHUNK
        nch = (e1 - a0 + CHUNK - 1) // CHUNK

        def stage_start(c, s):
            base = a0 + c * CHUNK
            pltpu.async_copy(src.at[pl.ds(base, CHUNK)], src_v[s], semS[s])
            pltpu.async_copy(dst.at[pl.ds(base, CHUNK)], dst_v[s], semS[s])
            pltpu.async_copy(w.at[pl.ds(base, CHUNK)], w_v[s], semS[s])

        def stage_wait(s):
            pltpu.make_async_copy(src.at[pl.ds(0, CHUNK)], src_v[s], semS[s]).wait()
            pltpu.make_async_copy(dst.at[pl.ds(0, CHUNK)], dst_v[s], semS[s]).wait()
            pltpu.make_async_copy(w.at[pl.ds(0, CHUNK)], w_v[s], semS[s]).wait()

        def gather_start(s):
            pltpu.async_copy(table.at[src_v[s]], rows_v[s], semG[s])

        def gather_wait(s):
            pltpu.make_async_copy(table.at[src_v[s]], rows_v[s], semG[s]).wait()

        def compute(c, s):
            base = a0 + c * CHUNK
            jlo = jnp.maximum(e0 - base, 0)
            jhi = jnp.minimum(e1 - base, CHUNK)
            iota = lax.iota(jnp.int32, 16)

            def group(g, carry):
                lane = iota + g * 16
                m = (lane >= jlo) & (lane < jhi)
                dvec = dst_v[s][pl.ds(g * 16, 16)]
                wvec = w_v[s][pl.ds(g * 16, 16)]
                wm = jnp.where(m, wvec, 0.0)
                dl = jnp.clip(dvec - row0, 0, rows_per_tile - 1)
                for l in range(16):
                    wj = wm[l]
                    dlj = dl[l]
                    r = g * 16 + l

                    @plsc.parallel_loop(0, F // 16, 1, unroll=8)
                    def _(k, _wj=wj, _dlj=dlj, _r=r):
                        plsc.addupdate(
                            acc_v.at[_dlj, pl.ds(k * 16, 16)],
                            _wj * rows_v[s][_r, pl.ds(k * 16, 16)],
                        )
                return carry

            lax.fori_loop(0, CHUNK // 16, group, 0)

        @pl.when(nch > 0)
        def _():
            stage_start(0, 0)

        zero = jnp.zeros((16,), jnp.float32)

        def zrow(r, carry):
            for k in range(F // 16):
                acc_v[r, pl.ds(k * 16, 16)] = zero
            return carry

        lax.fori_loop(0, rows_per_tile, zrow, 0)

        @pl.when(nch > 0)
        def _():
            stage_wait(0)
            gather_start(0)

        @pl.when(nch > 1)
        def _():
            stage_start(1, 1)

        def pair_body(i, carry):
            c0 = 2 * i
            c1 = c0 + 1
            c2 = c0 + 2
            c3 = c0 + 3

            @pl.when(c1 < nch)
            def _():
                stage_wait(1)
                gather_start(1)

            gather_wait(0)
            compute(c0, 0)

            @pl.when(c2 < nch)
            def _():
                stage_start(c2, 0)
                stage_wait(0)
                gather_start(0)

            @pl.when(c1 < nch)
            def _():
                gather_wait(1)
                compute(c1, 1)

            @pl.when(c3 < nch)
            def _():
                stage_start(c3, 1)

            return carry

        lax.fori_loop(0, (nch + 1) // 2, pair_body, 0)

        pltpu.sync_copy(acc_v, out.at[pl.ds(row0, rows_per_tile)])

    return spmm


def _round8(v):
    return (v + 7) // 8 * 8


ROWS_D = _round8((N_TRUNC + NW - 1) // NW)
ROWS_U = _round8((N_FULL + NW - 1) // NW)

_spmm_down = _make_spmm(ROWS_D, 128)
_spmm_up = _make_spmm(ROWS_U, 64)


def _bounds(dst, rows_per_tile):
    edges = jnp.arange(NW + 1, dtype=jnp.int32) * rows_per_tile
    b = jnp.searchsorted(dst, edges, side="left").astype(jnp.int32)
    return jnp.concatenate([b, jnp.full((NB - NW - 1,), E, jnp.int32)])


def kernel(x, down_src, down_dst, down_w, up_src, up_dst, up_w):
    batch, _, ens, _, _ = x.shape
    xb = x[:, -1].reshape(batch * ens, N_FULL, D_FEAT)
    xt = jnp.transpose(xb, (1, 0, 2)).reshape(N_FULL, F)

    bd = _bounds(down_dst, ROWS_D)
    bu = _bounds(up_dst, ROWS_U)

    h = _spmm_down(xt, down_src, down_dst, down_w, bd)
    y = _spmm_up(h, up_src, up_dst, up_w, bu)

    y = y[:N_FULL].reshape(N_FULL, BE, D_FEAT)
    return jnp.transpose(y, (1, 0, 2)).reshape(batch, ens, N_FULL, D_FEAT)


# speedup vs baseline: 52.3582x; 1.0687x over previous
import functools

import jax
import jax.numpy as jnp
from jax import lax
from jax.experimental import pallas as pl
from jax.experimental.pallas import tpu as pltpu
from jax.experimental.pallas import tpu_sc as plsc

N_FULL = 10000
N_TRUNC = 2500
E = 160000
D_FEAT = 128
BE = 2
F = BE * D_FEAT
NW = 32
NB = 48


def _make_spmm(rows_per_tile, CHUNK, split_table=False):
    mesh = plsc.VectorSubcoreMesh(core_axis_name="c", subcore_axis_name="s")

    @functools.partial(
        pl.kernel,
        out_type=jax.ShapeDtypeStruct((NW * rows_per_tile, F), jnp.float32),
        mesh=mesh,
        scratch_types=[
            pltpu.VMEM((rows_per_tile, F), jnp.float32),
            [[pltpu.VMEM((CHUNK, D_FEAT), jnp.float32)] * BE
             if split_table else
             pltpu.VMEM((CHUNK, F), jnp.float32)] * 2,
            [pltpu.VMEM((CHUNK,), jnp.int32)] * 2,
            [pltpu.VMEM((CHUNK,), jnp.int32)] * 2,
            [pltpu.VMEM((CHUNK,), jnp.float32)] * 2,
            pltpu.VMEM((NB,), jnp.int32),
            [pltpu.SemaphoreType.DMA] * 2,
            [pltpu.SemaphoreType.DMA] * 2,
        ],
    )
    def spmm(table, src, dst, w, bounds, out, acc_v, rows_v, src_v, dst_v,
             w_v, b_v, semS, semG):
        wid = lax.axis_index("c") * 16 + lax.axis_index("s")
        row0 = wid * rows_per_tile

        pltpu.sync_copy(bounds, b_v)
        bb = b_v[pl.ds(wid, 16)]
        e0 = bb[0]
        e1 = bb[1]

        a0 = (e0 // CHUNK) * CHUNK
        nch = (e1 - a0 + CHUNK - 1) // CHUNK

        def stage_start(c, s):
            base = a0 + c * CHUNK
            pltpu.async_copy(src.at[pl.ds(base, CHUNK)], src_v[s], semS[s])
            pltpu.async_copy(dst.at[pl.ds(base, CHUNK)], dst_v[s], semS[s])
            pltpu.async_copy(w.at[pl.ds(base, CHUNK)], w_v[s], semS[s])

        def stage_wait(s):
            pltpu.make_async_copy(src.at[pl.ds(0, CHUNK)], src_v[s], semS[s]).wait()
            pltpu.make_async_copy(dst.at[pl.ds(0, CHUNK)], dst_v[s], semS[s]).wait()
            pltpu.make_async_copy(w.at[pl.ds(0, CHUNK)], w_v[s], semS[s]).wait()

        def gather_start(s):
            if split_table:
                for b in range(BE):
                    pltpu.async_copy(table.at[b, 1].at[src_v[s]],
                                     rows_v[s][b], semG[s])
            else:
                pltpu.async_copy(table.at[src_v[s]], rows_v[s], semG[s])

        def gather_wait(s):
            if split_table:
                for b in range(BE):
                    pltpu.make_async_copy(table.at[b, 1].at[src_v[s]],
                                          rows_v[s][b], semG[s]).wait()
            else:
                pltpu.make_async_copy(table.at[src_v[s]], rows_v[s], semG[s]).wait()

        def compute(c, s):
            base = a0 + c * CHUNK
            jlo = jnp.maximum(e0 - base, 0)
            jhi = jnp.minimum(e1 - base, CHUNK)
            iota = lax.iota(jnp.int32, 16)

            def group(g, carry):
                lane = iota + g * 16
                m = (lane >= jlo) & (lane < jhi)
                dvec = dst_v[s][pl.ds(g * 16, 16)]
                wvec = w_v[s][pl.ds(g * 16, 16)]
                wm = jnp.where(m, wvec, 0.0)
                dl = jnp.clip(dvec - row0, 0, rows_per_tile - 1)
                for l in range(16):
                    wj = wm[l]
                    dlj = dl[l]
                    r = g * 16 + l

                    if split_table:
                        for b in range(BE):
                            @plsc.parallel_loop(0, D_FEAT // 16, 1, unroll=8)
                            def _(k, _wj=wj, _dlj=dlj, _r=r, _b=b):
                                plsc.addupdate(
                                    acc_v.at[_dlj,
                                             pl.ds(_b * D_FEAT + k * 16, 16)],
                                    _wj * rows_v[s][_b][_r, pl.ds(k * 16, 16)],
                                )
                    else:
                        @plsc.parallel_loop(0, F // 16, 1, unroll=8)
                        def _(k, _wj=wj, _dlj=dlj, _r=r):
                            plsc.addupdate(
                                acc_v.at[_dlj, pl.ds(k * 16, 16)],
                                _wj * rows_v[s][_r, pl.ds(k * 16, 16)],
                            )
                return carry

            lax.fori_loop(0, CHUNK // 16, group, 0)

        @pl.when(nch > 0)
        def _():
            stage_start(0, 0)

        zero = jnp.zeros((16,), jnp.float32)

        def zrow(r, carry):
            for k in range(F // 16):
                acc_v[r, pl.ds(k * 16, 16)] = zero
            return carry

        lax.fori_loop(0, rows_per_tile, zrow, 0)

        @pl.when(nch > 0)
        def _():
            stage_wait(0)
            gather_start(0)

        @pl.when(nch > 1)
        def _():
            stage_start(1, 1)

        def pair_body(i, carry):
            c0 = 2 * i
            c1 = c0 + 1
            c2 = c0 + 2
            c3 = c0 + 3

            @pl.when(c1 < nch)
            def _():
                stage_wait(1)
                gather_start(1)

            gather_wait(0)
            compute(c0, 0)

            @pl.when(c2 < nch)
            def _():
                stage_start(c2, 0)
                stage_wait(0)
                gather_start(0)

            @pl.when(c1 < nch)
            def _():
                gather_wait(1)
                compute(c1, 1)

            @pl.when(c3 < nch)
            def _():
                stage_start(c3, 1)

            return carry

        lax.fori_loop(0, (nch + 1) // 2, pair_body, 0)

        pltpu.sync_copy(acc_v, out.at[pl.ds(row0, rows_per_tile)])

    return spmm


def _round8(v):
    return (v + 7) // 8 * 8


ROWS_D = _round8((N_TRUNC + NW - 1) // NW)
ROWS_U = _round8((N_FULL + NW - 1) // NW)

_spmm_down = _make_spmm(ROWS_D, 128, split_table=True)
_spmm_up = _make_spmm(ROWS_U, 64)


def _bounds(dst, rows_per_tile):
    edges = jnp.arange(NW + 1, dtype=jnp.int32) * rows_per_tile
    b = jnp.searchsorted(dst, edges, side="left").astype(jnp.int32)
    return jnp.concatenate([b, jnp.full((NB - NW - 1,), E, jnp.int32)])


def kernel(x, down_src, down_dst, down_w, up_src, up_dst, up_w):
    batch, nt, ens, _, _ = x.shape
    xr = x.reshape(batch * ens, nt, N_FULL, D_FEAT)

    bd = _bounds(down_dst, ROWS_D)
    bu = _bounds(up_dst, ROWS_U)

    h = _spmm_down(xr, down_src, down_dst, down_w, bd)
    y = _spmm_up(h, up_src, up_dst, up_w, bu)

    y = y[:N_FULL].reshape(N_FULL, BE, D_FEAT)
    return jnp.transpose(y, (1, 0, 2)).reshape(batch, ens, N_FULL, D_FEAT)
